# baseline (device time: 100881 ns/iter reference)
import jax
import jax.numpy as jnp
from jax import lax
from jax.experimental import pallas as pl
from jax.experimental.pallas import tpu as pltpu

N_DEV = 8
N_Q = 4


def _body(my_ref, x_ref, w_ref, out_ref, send_ref, recv_ref, amax_scr,
          my_amax, stage_ref, send_sems, recv_sems, amax_send_sems,
          amax_recv_sems, copy_sem):
    s = pl.program_id(0)
    t = s // N_Q
    q = s % N_Q
    del my_ref
    my = lax.axis_index("i")
    j = lax.rem(my + t, N_DEV)
    n_q = w_ref.shape[1]
    m_per = x_ref.shape[0]

    wb = w_ref[...].astype(jnp.bfloat16)
    y = jnp.dot(x_ref[...], wb, preferred_element_type=jnp.float32)
    y = jnp.maximum(y, 0.0)
    mx = jnp.max(y)

    @pl.when(s == 0)
    def _():
        my_amax[...] = jnp.broadcast_to(mx, (1, 128))

    @pl.when(s > 0)
    def _():
        my_amax[...] = jnp.maximum(my_amax[...], mx)

    send_ref[j, :, pl.ds(q * n_q, n_q)] = y.astype(jnp.bfloat16)

    pltpu.make_async_remote_copy(
        src_ref=send_ref.at[j, :, pl.ds(q * n_q, n_q)],
        dst_ref=recv_ref.at[my, :, pl.ds(q * n_q, n_q)],
        send_sem=send_sems.at[j, q],
        recv_sem=recv_sems.at[my, q],
        device_id=(j,),
        device_id_type=pl.DeviceIdType.MESH,
    ).start()

    @pl.when(s == N_DEV * N_Q - 1)
    def _():
        for d in range(N_DEV):
            pltpu.make_async_remote_copy(
                src_ref=my_amax,
                dst_ref=amax_scr.at[my],
                send_sem=amax_send_sems.at[d],
                recv_sem=amax_recv_sems.at[my],
                device_id=(d,),
                device_id_type=pl.DeviceIdType.MESH,
            ).start()
        for i in range(N_DEV):
            adesc = pltpu.make_async_remote_copy(
                src_ref=my_amax, dst_ref=amax_scr.at[i],
                send_sem=amax_send_sems.at[i], recv_sem=amax_recv_sems.at[i],
                device_id=(my,), device_id_type=pl.DeviceIdType.MESH,
            )
            adesc.wait_recv()
            adesc.wait_send()

        gmax = jnp.max(amax_scr[...])
        scale = gmax * (1.0 / 448.0)
        inv = 448.0 / jnp.maximum(gmax, 1e-30)

        for k in range(N_DEV):
            i = lax.rem(my - k + N_DEV, N_DEV)
            for qq in range(N_Q):
                desc = pltpu.make_async_remote_copy(
                    src_ref=send_ref.at[i, :, pl.ds(qq * n_q, n_q)],
                    dst_ref=recv_ref.at[i, :, pl.ds(qq * n_q, n_q)],
                    send_sem=send_sems.at[i, qq],
                    recv_sem=recv_sems.at[i, qq],
                    device_id=(my,), device_id_type=pl.DeviceIdType.MESH,
                )
                desc.wait_recv()
            yv = recv_ref[i].astype(jnp.float32)
            qv = jnp.minimum(yv * inv, 448.0).astype(jnp.float8_e4m3fn)
            stage_ref[...] = qv.astype(jnp.float32) * scale
            cp = pltpu.make_async_copy(
                stage_ref, out_ref.at[pl.ds(i * m_per, m_per), :], copy_sem)
            cp.start()
            cp.wait()

        for i in range(N_DEV):
            for qq in range(N_Q):
                pltpu.make_async_remote_copy(
                    src_ref=send_ref.at[i, :, pl.ds(qq * n_q, n_q)],
                    dst_ref=recv_ref.at[i, :, pl.ds(qq * n_q, n_q)],
                    send_sem=send_sems.at[i, qq],
                    recv_sem=recv_sems.at[i, qq],
                    device_id=(my,), device_id_type=pl.DeviceIdType.MESH,
                ).wait_send()


def kernel(x, w_mat):
    m_per, k = x.shape
    n = w_mat.shape[1]
    n_per = n // N_DEV
    n_q = n_per // N_Q

    xb = x.astype(jnp.bfloat16)
    my_idx = lax.axis_index("i").astype(jnp.int32).reshape((1,))

    def w_index(s, my_ref):
        t = s // N_Q
        q = s % N_Q
        j = lax.rem(my_ref[0] + t, N_DEV)
        return (0, j * N_Q + q)

    out = pl.pallas_call(
        _body,
        grid_spec=pltpu.PrefetchScalarGridSpec(
            num_scalar_prefetch=1,
            grid=(N_DEV * N_Q,),
            in_specs=[
                pl.BlockSpec((m_per, k), lambda s, my_ref: (0, 0),
                             memory_space=pltpu.VMEM),
                pl.BlockSpec((k, n_q), w_index,
                             memory_space=pltpu.VMEM),
            ],
            out_specs=pl.BlockSpec(memory_space=pl.ANY),
            scratch_shapes=[
                pltpu.VMEM((N_DEV, m_per, n_per), jnp.bfloat16),
                pltpu.VMEM((N_DEV, m_per, n_per), jnp.bfloat16),
                pltpu.VMEM((N_DEV, 1, 128), jnp.float32),
                pltpu.VMEM((1, 128), jnp.float32),
                pltpu.VMEM((m_per, n_per), jnp.float32),
                pltpu.SemaphoreType.DMA((N_DEV, N_Q)),
                pltpu.SemaphoreType.DMA((N_DEV, N_Q)),
                pltpu.SemaphoreType.DMA((N_DEV,)),
                pltpu.SemaphoreType.DMA((N_DEV,)),
                pltpu.SemaphoreType.DMA,
            ],
        ),
        out_shape=jax.ShapeDtypeStruct((N_DEV * m_per, n_per), jnp.float32),
    )(my_idx, xb, w_mat)
    return out


# device time: 96189 ns/iter; 1.0488x vs baseline; 1.0488x over previous
import jax
import jax.numpy as jnp
from jax import lax
from jax.experimental import pallas as pl
from jax.experimental.pallas import tpu as pltpu

N_DEV = 8
N_HALF = 2
N_SPLIT = 2


def _a2a_body(my_ref, x_ref, w_ref, y_ref, amax_ref, send_ref, my_amax,
              send_sems, recv_sems, amax_send_sems, amax_recv_sems):
    s = pl.program_id(0)
    t = s // N_HALF
    h = s % N_HALF
    del my_ref
    my = lax.axis_index("i")
    j = lax.rem(my + t, N_DEV)
    n_half = w_ref.shape[1]
    n_q = n_half // N_SPLIT

    wb = w_ref[...].astype(jnp.bfloat16)
    y = jnp.dot(x_ref[...], wb, preferred_element_type=jnp.float32)
    y = jnp.maximum(y, 0.0)
    mx = jnp.max(y)

    @pl.when(s == 0)
    def _():
        my_amax[...] = jnp.broadcast_to(mx, (1, 128))

    @pl.when(s > 0)
    def _():
        my_amax[...] = jnp.maximum(my_amax[...], mx)

    send_ref[j, :, pl.ds(h * n_half, n_half)] = y.astype(jnp.bfloat16)

    for sp in range(N_SPLIT):
        col = pl.ds(h * n_half + sp * n_q, n_q)
        pltpu.make_async_remote_copy(
            src_ref=send_ref.at[j, :, col],
            dst_ref=y_ref.at[my, :, col],
            send_sem=send_sems.at[j, h, sp],
            recv_sem=recv_sems.at[my, h, sp],
            device_id=(j,),
            device_id_type=pl.DeviceIdType.MESH,
        ).start()

    @pl.when(s == N_DEV * N_HALF - 1)
    def _():
        for d in range(N_DEV):
            pltpu.make_async_remote_copy(
                src_ref=my_amax,
                dst_ref=amax_ref.at[my],
                send_sem=amax_send_sems.at[d],
                recv_sem=amax_recv_sems.at[my],
                device_id=(d,),
                device_id_type=pl.DeviceIdType.MESH,
            ).start()
        for i in range(N_DEV):
            for hh in range(N_HALF):
                for sp in range(N_SPLIT):
                    col = pl.ds(hh * n_half + sp * n_q, n_q)
                    desc = pltpu.make_async_remote_copy(
                        src_ref=send_ref.at[i, :, col],
                        dst_ref=y_ref.at[i, :, col],
                        send_sem=send_sems.at[i, hh, sp],
                        recv_sem=recv_sems.at[i, hh, sp],
                        device_id=(my,), device_id_type=pl.DeviceIdType.MESH,
                    )
                    desc.wait_recv()
                    desc.wait_send()
            adesc = pltpu.make_async_remote_copy(
                src_ref=my_amax, dst_ref=amax_ref.at[i],
                send_sem=amax_send_sems.at[i], recv_sem=amax_recv_sems.at[i],
                device_id=(my,), device_id_type=pl.DeviceIdType.MESH,
            )
            adesc.wait_recv()
            adesc.wait_send()


def _quant_body(amax_ref, y_ref, out_ref):
    gmax = jnp.max(amax_ref[...])
    scale = gmax * (1.0 / 448.0)
    inv = 448.0 / jnp.maximum(gmax, 1e-30)
    yf = y_ref[0].astype(jnp.float32)
    q = jnp.minimum(yf * inv, 448.0).astype(jnp.float8_e4m3fn)
    out_ref[...] = (q.astype(jnp.float32) * scale).astype(jnp.bfloat16)


def kernel(x, w_mat):
    m_per, k = x.shape
    n = w_mat.shape[1]
    n_per = n // N_DEV
    n_half = n_per // N_HALF

    xb = x.astype(jnp.bfloat16)
    my_idx = lax.axis_index("i").astype(jnp.int32).reshape((1,))

    def w_index(s, my_ref):
        t = s // N_HALF
        h = s % N_HALF
        j = lax.rem(my_ref[0] + t, N_DEV)
        return (0, j * N_HALF + h)

    y_a2a, amax = pl.pallas_call(
        _a2a_body,
        grid_spec=pltpu.PrefetchScalarGridSpec(
            num_scalar_prefetch=1,
            grid=(N_DEV * N_HALF,),
            in_specs=[
                pl.BlockSpec((m_per, k), lambda s, my_ref: (0, 0),
                             memory_space=pltpu.VMEM),
                pl.BlockSpec((k, n_half), w_index,
                             memory_space=pltpu.VMEM),
            ],
            out_specs=[
                pl.BlockSpec(memory_space=pl.ANY),
                pl.BlockSpec(memory_space=pl.ANY),
            ],
            scratch_shapes=[
                pltpu.VMEM((N_DEV, m_per, n_per), jnp.bfloat16),
                pltpu.VMEM((1, 128), jnp.float32),
                pltpu.SemaphoreType.DMA((N_DEV, N_HALF, N_SPLIT)),
                pltpu.SemaphoreType.DMA((N_DEV, N_HALF, N_SPLIT)),
                pltpu.SemaphoreType.DMA((N_DEV,)),
                pltpu.SemaphoreType.DMA((N_DEV,)),
            ],
        ),
        out_shape=[
            jax.ShapeDtypeStruct((N_DEV, m_per, n_per), jnp.bfloat16),
            jax.ShapeDtypeStruct((N_DEV, 1, 128), jnp.float32),
        ],
    )(my_idx, xb, w_mat)

    out = pl.pallas_call(
        _quant_body,
        grid=(N_DEV,),
        in_specs=[
            pl.BlockSpec((N_DEV, 1, 128), lambda i: (0, 0, 0),
                         memory_space=pltpu.VMEM),
            pl.BlockSpec((1, m_per, n_per), lambda i: (i, 0, 0),
                         memory_space=pltpu.VMEM),
        ],
        out_specs=pl.BlockSpec((m_per, n_per), lambda i: (i, 0),
                               memory_space=pltpu.VMEM),
        out_shape=jax.ShapeDtypeStruct((N_DEV * m_per, n_per), jnp.bfloat16),
    )(amax, y_a2a)
    return out
